# initial kernel scaffold (unmeasured)
import jax
import jax.numpy as jnp
from jax import lax
from jax.experimental import pallas as pl
from jax.experimental.pallas import tpu as pltpu

N_DEV = 32
E_LOCAL = 4
N_TOK = 1024
D_MODEL = 256
D_FF = 512
CHUNK = N_TOK // N_DEV


def kernel(x, router_W, route_idx, expert_W):
    def body(
        x_ref,
        rw_ref,
        idx_ref,
        ew_ref,
        out_ref,
        partial_ref,
        rs_recv_ref,
        rs_send_sems,
        rs_recv_sems,
        ag_send_sems,
        ag_recv_sems,
    ):
        my = lax.axis_index("i")
        right = lax.rem(my + 1, N_DEV)

        xv = x_ref[...]
        scores = jnp.dot(xv, rw_ref[...], preferred_element_type=jnp.float32)
        p = jnp.exp(scores - jnp.max(scores, axis=-1, keepdims=True))
        e0 = idx_ref[:, 0:1]
        e1 = idx_ref[:, 1:2]
        lanes = lax.broadcasted_iota(jnp.int32, scores.shape, 1)
        g0 = jnp.sum(jnp.where(lanes == e0, p, 0.0), axis=-1, keepdims=True)
        g1 = jnp.sum(jnp.where(lanes == e1, p, 0.0), axis=-1, keepdims=True)
        gs = g0 + g1
        w0 = g0 / gs
        w1 = g1 / gs

        acc = jnp.zeros((N_TOK, D_FF), jnp.float32)
        for j in range(E_LOCAL):
            ge = my * E_LOCAL + j
            wj = jnp.where(e0 == ge, w0, 0.0) + jnp.where(e1 == ge, w1, 0.0)
            acc = acc + jnp.dot(
                xv * wj, ew_ref[j], preferred_element_type=jnp.float32
            )
        partial_ref[...] = acc

        for s in range(N_DEV - 1):
            c_send = lax.rem(my - s + N_DEV, N_DEV)
            rdma = pltpu.make_async_remote_copy(
                src_ref=partial_ref.at[pl.ds(c_send * CHUNK, CHUNK), :],
                dst_ref=rs_recv_ref.at[s],
                send_sem=rs_send_sems.at[s],
                recv_sem=rs_recv_sems.at[s],
                device_id=(right,),
                device_id_type=pl.DeviceIdType.MESH,
            )
            rdma.start()
            rdma.wait()
            c_recv = lax.rem(my - s - 1 + N_DEV, N_DEV)
            sl = pl.ds(c_recv * CHUNK, CHUNK)
            partial_ref[sl, :] = partial_ref[sl, :] + rs_recv_ref[s]

        c_own = lax.rem(my + 1, N_DEV)
        sl_own = pl.ds(c_own * CHUNK, CHUNK)
        out_ref[sl_own, :] = partial_ref[sl_own, :]

        for t in range(N_DEV - 1):
            c = lax.rem(my + 1 - t + N_DEV, N_DEV)
            sl = pl.ds(c * CHUNK, CHUNK)
            rdma = pltpu.make_async_remote_copy(
                src_ref=out_ref.at[sl, :],
                dst_ref=out_ref.at[sl, :],
                send_sem=ag_send_sems.at[t],
                recv_sem=ag_recv_sems.at[t],
                device_id=(right,),
                device_id_type=pl.DeviceIdType.MESH,
            )
            rdma.start()
            rdma.wait()

    return pl.pallas_call(
        body,
        out_shape=jax.ShapeDtypeStruct((N_TOK, D_FF), jnp.float32),
        in_specs=[
            pl.BlockSpec(memory_space=pltpu.VMEM),
            pl.BlockSpec(memory_space=pltpu.VMEM),
            pl.BlockSpec(memory_space=pltpu.VMEM),
            pl.BlockSpec(memory_space=pltpu.VMEM),
        ],
        out_specs=pl.BlockSpec(memory_space=pltpu.VMEM),
        scratch_shapes=[
            pltpu.VMEM((N_TOK, D_FF), jnp.float32),
            pltpu.VMEM((N_DEV - 1, CHUNK, D_FF), jnp.float32),
            pltpu.SemaphoreType.DMA((N_DEV - 1,)),
            pltpu.SemaphoreType.DMA((N_DEV - 1,)),
            pltpu.SemaphoreType.DMA((N_DEV - 1,)),
            pltpu.SemaphoreType.DMA((N_DEV - 1,)),
        ],
        compiler_params=pltpu.CompilerParams(collective_id=0),
    )(x, router_W, route_idx, expert_W)


# baseline (device time: 179891 ns/iter reference)
import jax
import jax.numpy as jnp
from jax import lax
from jax.experimental import pallas as pl
from jax.experimental.pallas import tpu as pltpu

N_DEV = 32
E_LOCAL = 4
N_TOK = 1024
D_MODEL = 256
D_FF = 512
CHUNK = N_TOK // N_DEV


def kernel(x, router_W, route_idx, expert_W):
    def body(
        x_ref,
        rw_ref,
        idx_ref,
        ew_ref,
        out_ref,
        partial_ref,
        rs_recv_ref,
        rs_send_sems,
        rs_recv_sems,
        ag_send_sems,
        ag_recv_sems,
    ):
        my = lax.axis_index("i")
        right = lax.rem(my + 1, N_DEV)

        xv = x_ref[...]
        scores = jnp.dot(xv, rw_ref[...], preferred_element_type=jnp.float32)
        p = jnp.exp(scores - jnp.max(scores, axis=-1, keepdims=True))
        e0 = idx_ref[:, 0:1]
        e1 = idx_ref[:, 1:2]
        lanes = lax.broadcasted_iota(jnp.int32, scores.shape, 1)
        g0 = jnp.sum(jnp.where(lanes == e0, p, 0.0), axis=-1, keepdims=True)
        g1 = jnp.sum(jnp.where(lanes == e1, p, 0.0), axis=-1, keepdims=True)
        gs = g0 + g1
        w0 = g0 / gs
        w1 = g1 / gs

        acc = jnp.zeros((N_TOK, D_FF), jnp.float32)
        for j in range(E_LOCAL):
            ge = my * E_LOCAL + j
            wj = jnp.where(e0 == ge, w0, 0.0) + jnp.where(e1 == ge, w1, 0.0)
            acc = acc + jnp.dot(
                xv * wj, ew_ref[j], preferred_element_type=jnp.float32
            )
        partial_ref[...] = acc

        for s in range(N_DEV - 1):
            c_send = lax.rem(my - s + N_DEV, N_DEV)
            rdma = pltpu.make_async_remote_copy(
                src_ref=partial_ref.at[pl.ds(c_send * CHUNK, CHUNK), :],
                dst_ref=rs_recv_ref.at[s],
                send_sem=rs_send_sems.at[s],
                recv_sem=rs_recv_sems.at[s],
                device_id=(right,),
                device_id_type=pl.DeviceIdType.MESH,
            )
            rdma.start()
            rdma.wait()
            c_recv = lax.rem(my - s - 1 + N_DEV, N_DEV)
            sl = pl.ds(c_recv * CHUNK, CHUNK)
            partial_ref[sl, :] = partial_ref[sl, :] + rs_recv_ref[s]

        c_own = lax.rem(my + 1, N_DEV)
        sl_own = pl.ds(c_own * CHUNK, CHUNK)
        out_ref[sl_own, :] = partial_ref[sl_own, :]

        for t in range(N_DEV - 1):
            c = lax.rem(my + 1 - t + N_DEV, N_DEV)
            sl = pl.ds(c * CHUNK, CHUNK)
            rdma = pltpu.make_async_remote_copy(
                src_ref=out_ref.at[sl, :],
                dst_ref=out_ref.at[sl, :],
                send_sem=ag_send_sems.at[t],
                recv_sem=ag_recv_sems.at[t],
                device_id=(right,),
                device_id_type=pl.DeviceIdType.MESH,
            )
            rdma.start()
            rdma.wait()

    return pl.pallas_call(
        body,
        out_shape=jax.ShapeDtypeStruct((N_TOK, D_FF), jnp.float32),
        in_specs=[
            pl.BlockSpec(memory_space=pltpu.VMEM),
            pl.BlockSpec(memory_space=pltpu.VMEM),
            pl.BlockSpec(memory_space=pltpu.VMEM),
            pl.BlockSpec(memory_space=pltpu.VMEM),
        ],
        out_specs=pl.BlockSpec(memory_space=pltpu.VMEM),
        scratch_shapes=[
            pltpu.VMEM((N_TOK, D_FF), jnp.float32),
            pltpu.VMEM((N_DEV - 1, CHUNK, D_FF), jnp.float32),
            pltpu.SemaphoreType.DMA((N_DEV - 1,)),
            pltpu.SemaphoreType.DMA((N_DEV - 1,)),
            pltpu.SemaphoreType.DMA((N_DEV - 1,)),
            pltpu.SemaphoreType.DMA((N_DEV - 1,)),
        ],
    )(x, router_W, route_idx, expert_W)


# device time: 83594 ns/iter; 2.1520x vs baseline; 2.1520x over previous
import jax
import jax.numpy as jnp
from jax import lax
from jax.experimental import pallas as pl
from jax.experimental.pallas import tpu as pltpu

N_DEV = 32
E_LOCAL = 4
N_TOK = 1024
D_MODEL = 256
D_FF = 512

BITS = (0, 3, 1, 2, 4)
HALVES = tuple(N_TOK >> (k + 1) for k in range(5))
SCRATCH_OFFS = (0, 512, 768, 896, 960)
SCRATCH_ROWS = 992


def kernel(x, router_W, route_idx, expert_W):
    def body(
        x_ref,
        rw_ref,
        idx_ref,
        ew_ref,
        out_ref,
        partial_ref,
        scratch_ref,
        rs_send_sems,
        rs_recv_sems,
        ag_send_sems,
        ag_recv_sems,
    ):
        my = lax.axis_index("i")

        xv = x_ref[...]
        scores = jnp.dot(xv, rw_ref[...], preferred_element_type=jnp.float32)
        p = jnp.exp(scores - jnp.max(scores, axis=-1, keepdims=True))
        e0 = idx_ref[:, 0:1]
        e1 = idx_ref[:, 1:2]
        lanes = lax.broadcasted_iota(jnp.int32, scores.shape, 1)
        g0 = jnp.sum(jnp.where(lanes == e0, p, 0.0), axis=-1, keepdims=True)
        g1 = jnp.sum(jnp.where(lanes == e1, p, 0.0), axis=-1, keepdims=True)
        gs = g0 + g1
        w0 = g0 / gs
        w1 = g1 / gs

        acc = jnp.zeros((N_TOK, D_FF), jnp.float32)
        for j in range(E_LOCAL):
            ge = my * E_LOCAL + j
            wj = jnp.where(e0 == ge, w0, 0.0) + jnp.where(e1 == ge, w1, 0.0)
            acc = acc + jnp.dot(
                xv * wj, ew_ref[j], preferred_element_type=jnp.float32
            )
        partial_ref[...] = acc

        off = jnp.int32(0)
        parent_offs = []
        for k in range(5):
            bit = BITS[k]
            half = HALVES[k]
            s_off = SCRATCH_OFFS[k]
            partner = my ^ (1 << bit)
            b = (my >> bit) & 1
            send_off = off + jnp.where(b == 0, half, 0)
            keep_off = off + b * half
            rdma = pltpu.make_async_remote_copy(
                src_ref=partial_ref.at[pl.ds(send_off, half), :],
                dst_ref=scratch_ref.at[pl.ds(s_off, half), :],
                send_sem=rs_send_sems.at[k],
                recv_sem=rs_recv_sems.at[k],
                device_id=(partner,),
                device_id_type=pl.DeviceIdType.MESH,
            )
            rdma.start()
            rdma.wait()
            sl = pl.ds(keep_off, half)
            partial_ref[sl, :] = (
                partial_ref[sl, :] + scratch_ref[pl.ds(s_off, half), :]
            )
            parent_offs.append(off)
            off = keep_off

        sl_own = pl.ds(off, HALVES[4])
        out_ref[sl_own, :] = partial_ref[sl_own, :]

        own_off = off
        for k in [4, 3, 2, 1, 0]:
            bit = BITS[k]
            size = HALVES[k]
            partner = my ^ (1 << bit)
            sl = pl.ds(own_off, size)
            rdma = pltpu.make_async_remote_copy(
                src_ref=out_ref.at[sl, :],
                dst_ref=out_ref.at[sl, :],
                send_sem=ag_send_sems.at[k],
                recv_sem=ag_recv_sems.at[k],
                device_id=(partner,),
                device_id_type=pl.DeviceIdType.MESH,
            )
            rdma.start()
            rdma.wait()
            own_off = parent_offs[k]

    return pl.pallas_call(
        body,
        out_shape=jax.ShapeDtypeStruct((N_TOK, D_FF), jnp.float32),
        in_specs=[
            pl.BlockSpec(memory_space=pltpu.VMEM),
            pl.BlockSpec(memory_space=pltpu.VMEM),
            pl.BlockSpec(memory_space=pltpu.VMEM),
            pl.BlockSpec(memory_space=pltpu.VMEM),
        ],
        out_specs=pl.BlockSpec(memory_space=pltpu.VMEM),
        scratch_shapes=[
            pltpu.VMEM((N_TOK, D_FF), jnp.float32),
            pltpu.VMEM((SCRATCH_ROWS, D_FF), jnp.float32),
            pltpu.SemaphoreType.DMA((5,)),
            pltpu.SemaphoreType.DMA((5,)),
            pltpu.SemaphoreType.DMA((5,)),
            pltpu.SemaphoreType.DMA((5,)),
        ],
    )(x, router_W, route_idx, expert_W)


# device time: 67242 ns/iter; 2.6753x vs baseline; 1.2432x over previous
import jax
import jax.numpy as jnp
from jax import lax
from jax.experimental import pallas as pl
from jax.experimental.pallas import tpu as pltpu

N_DEV = 32
E_LOCAL = 4
N_TOK = 1024
D_MODEL = 256
D_FF = 512

P_ROWS = N_TOK // 2
BITS_A = (0, 3, 1, 2, 4)
BITS_B = (3, 0, 2, 4, 1)
HALVES = tuple(P_ROWS >> (k + 1) for k in range(5))
SCR_A = (0, 256, 384, 448, 480)
SCR_B = tuple(o + 496 for o in SCR_A)
SCRATCH_ROWS = 992


def kernel(x, router_W, route_idx, expert_W):
    def body(
        x_ref,
        rw_ref,
        idx_ref,
        ew_ref,
        out_ref,
        partial_ref,
        scratch_ref,
        rsA_send,
        rsA_recv,
        rsB_send,
        rsB_recv,
        agA_send,
        agA_recv,
        agB_send,
        agB_recv,
    ):
        my = lax.axis_index("i")

        xv = x_ref[...]
        scores = jnp.dot(xv, rw_ref[...], preferred_element_type=jnp.float32)
        p = jnp.exp(scores - jnp.max(scores, axis=-1, keepdims=True))
        e0 = idx_ref[:, 0:1]
        e1 = idx_ref[:, 1:2]
        lanes = lax.broadcasted_iota(jnp.int32, scores.shape, 1)
        g0 = jnp.sum(jnp.where(lanes == e0, p, 0.0), axis=-1, keepdims=True)
        g1 = jnp.sum(jnp.where(lanes == e1, p, 0.0), axis=-1, keepdims=True)
        gs = g0 + g1
        w0 = g0 / gs
        w1 = g1 / gs

        acc = jnp.zeros((N_TOK, D_FF), jnp.float32)
        for j in range(E_LOCAL):
            ge = my * E_LOCAL + j
            wj = jnp.where(e0 == ge, w0, 0.0) + jnp.where(e1 == ge, w1, 0.0)
            acc = acc + jnp.dot(
                xv * wj, ew_ref[j], preferred_element_type=jnp.float32
            )
        partial_ref[...] = acc

        parts = [
            (BITS_A, SCR_A, rsA_send, rsA_recv, agA_send, agA_recv, 0),
            (BITS_B, SCR_B, rsB_send, rsB_recv, agB_send, agB_recv, P_ROWS),
        ]

        offs = [jnp.int32(p0) for *_, p0 in parts]
        parent_offs = [[], []]
        for k in range(5):
            half = HALVES[k]
            rdmas = []
            keep = []
            for i, (bits, scr, s_send, s_recv, _, _, _) in enumerate(parts):
                bit = bits[k]
                partner = my ^ (1 << bit)
                b = (my >> bit) & 1
                send_off = offs[i] + jnp.where(b == 0, half, 0)
                keep_off = offs[i] + b * half
                rdma = pltpu.make_async_remote_copy(
                    src_ref=partial_ref.at[pl.ds(send_off, half), :],
                    dst_ref=scratch_ref.at[pl.ds(scr[k], half), :],
                    send_sem=s_send.at[k],
                    recv_sem=s_recv.at[k],
                    device_id=(partner,),
                    device_id_type=pl.DeviceIdType.MESH,
                )
                rdma.start()
                rdmas.append(rdma)
                keep.append(keep_off)
            for i, (bits, scr, *_rest) in enumerate(parts):
                rdmas[i].wait()
                sl = pl.ds(keep[i], half)
                partial_ref[sl, :] = (
                    partial_ref[sl, :] + scratch_ref[pl.ds(scr[k], half), :]
                )
                parent_offs[i].append(offs[i])
                offs[i] = keep[i]

        for i in range(2):
            sl_own = pl.ds(offs[i], HALVES[4])
            out_ref[sl_own, :] = partial_ref[sl_own, :]

        own = list(offs)
        for k in reversed(range(5)):
            size = HALVES[k]
            rdmas = []
            for i, (bits, _, _, _, g_send, g_recv, _) in enumerate(parts):
                bit = bits[k]
                partner = my ^ (1 << bit)
                sl = pl.ds(own[i], size)
                rdma = pltpu.make_async_remote_copy(
                    src_ref=out_ref.at[sl, :],
                    dst_ref=out_ref.at[sl, :],
                    send_sem=g_send.at[k],
                    recv_sem=g_recv.at[k],
                    device_id=(partner,),
                    device_id_type=pl.DeviceIdType.MESH,
                )
                rdma.start()
                rdmas.append(rdma)
            for i in range(2):
                rdmas[i].wait()
                own[i] = parent_offs[i][k]

    return pl.pallas_call(
        body,
        out_shape=jax.ShapeDtypeStruct((N_TOK, D_FF), jnp.float32),
        in_specs=[
            pl.BlockSpec(memory_space=pltpu.VMEM),
            pl.BlockSpec(memory_space=pltpu.VMEM),
            pl.BlockSpec(memory_space=pltpu.VMEM),
            pl.BlockSpec(memory_space=pltpu.VMEM),
        ],
        out_specs=pl.BlockSpec(memory_space=pltpu.VMEM),
        scratch_shapes=[
            pltpu.VMEM((N_TOK, D_FF), jnp.float32),
            pltpu.VMEM((SCRATCH_ROWS, D_FF), jnp.float32),
            pltpu.SemaphoreType.DMA((5,)),
            pltpu.SemaphoreType.DMA((5,)),
            pltpu.SemaphoreType.DMA((5,)),
            pltpu.SemaphoreType.DMA((5,)),
            pltpu.SemaphoreType.DMA((5,)),
            pltpu.SemaphoreType.DMA((5,)),
            pltpu.SemaphoreType.DMA((5,)),
            pltpu.SemaphoreType.DMA((5,)),
        ],
    )(x, router_W, route_idx, expert_W)
